# baseline (speedup 1.0000x reference)
import jax
import jax.numpy as jnp
from jax import lax
from jax.experimental import pallas as pl
from jax.experimental.pallas import tpu as pltpu
from jax.experimental.pallas import tpu_sc as plsc

_N = 10000
_NP = 10240
_E = 320000
_D = 128
_B = 64
_C = 10

_NC = 2
_NS = 16
_KC = 80
_KG = 40
_EP = _NC * _NS * _KC * 128
_RPS = _NP // _NS

_BLK = 1024
_G = _NP // _BLK


def _sc_agg_body(h_hbm, src_hbm, dst_hbm, zero_hbm, out_hbm,
                 srcv, dstv, rows, acc, gs0, gs1):
    cid = lax.axis_index("c")
    sid = lax.axis_index("s")
    w = sid * _NC + cid
    gs = (gs0, gs1)

    def wait_gather(b):
        pltpu.make_async_copy(h_hbm.at[pl.ds(0, 128)], rows.at[b], gs[b]).wait()

    pltpu.sync_copy(zero_hbm, acc.at[pl.ds(sid * _RPS, _RPS)])
    plsc.subcore_barrier()

    for g in range(_KC // _KG):
        base = w * _KC + g * _KG
        pltpu.sync_copy(src_hbm.at[pl.ds(base, _KG)], srcv)
        pltpu.sync_copy(dst_hbm.at[pl.ds(base, _KG)], dstv)
        pltpu.async_copy(h_hbm.at[srcv.at[0]], rows.at[0], gs[0])

        def pair(j2, carry):
            for b in range(2):
                jj = 2 * j2 + b

                @pl.when(jj + 1 < _KG)
                def _():
                    pltpu.async_copy(h_hbm.at[srcv.at[jj + 1]],
                                     rows.at[1 - b], gs[1 - b])

                wait_gather(b)
                pltpu.sync_copy(rows.at[b], acc.at[dstv.at[jj]], add=True)
            return carry

        lax.fori_loop(0, _KG // 2, pair, 0)

    plsc.subcore_barrier()
    pltpu.sync_copy(acc.at[pl.ds(sid * _RPS, _RPS)],
                    out_hbm.at[cid, pl.ds(sid * _RPS, _RPS)])


def _make_sc_agg():
    return pl.kernel(
        _sc_agg_body,
        out_type=jax.ShapeDtypeStruct((_NC, _NP, _D), jnp.float32),
        mesh=plsc.VectorSubcoreMesh(core_axis_name="c", subcore_axis_name="s"),
        scratch_types=[
            pltpu.VMEM((_KG, 128), jnp.int32),
            pltpu.VMEM((_KG, 128), jnp.int32),
            pltpu.VMEM((2, 128, _D), jnp.float32),
            pltpu.VMEM_SHARED((_NP, _D), jnp.float32),
            pltpu.SemaphoreType.DMA,
            pltpu.SemaphoreType.DMA,
        ],
    )


def _mlp_body(eps_ref, agg_ref, h_ref, w1_ref, b1_ref, w2_ref, b2_ref,
              z2_ref, ssum_ref, ssq_ref, sacc, qacc):
    i = pl.program_id(0)
    out = agg_ref[0] + agg_ref[1] + (1.0 + eps_ref[0]) * h_ref[...]
    z1 = jnp.maximum(
        jnp.dot(out, w1_ref[...], preferred_element_type=jnp.float32)
        + b1_ref[...], 0.0)
    z2 = jnp.maximum(
        jnp.dot(z1, w2_ref[...], preferred_element_type=jnp.float32)
        + b2_ref[...], 0.0)
    z2_ref[...] = z2

    row = i * _BLK + lax.broadcasted_iota(jnp.int32, (_BLK, 1), 0)
    z2m = jnp.where(row < _N, z2, 0.0)

    @pl.when(i == 0)
    def _():
        sacc[...] = jnp.zeros_like(sacc)
        qacc[...] = jnp.zeros_like(qacc)

    sacc[...] += jnp.sum(z2m, axis=0, keepdims=True)
    qacc[...] += jnp.sum(z2m * z2m, axis=0, keepdims=True)

    @pl.when(i == pl.num_programs(0) - 1)
    def _():
        ssum_ref[...] = sacc[...]
        ssq_ref[...] = qacc[...]


_mlp_call = pl.pallas_call(
    _mlp_body,
    grid=(_G,),
    in_specs=[
        pl.BlockSpec(memory_space=pltpu.SMEM),
        pl.BlockSpec((_NC, _BLK, _D), lambda i: (0, i, 0)),
        pl.BlockSpec((_BLK, _D), lambda i: (i, 0)),
        pl.BlockSpec((_D, _D), lambda i: (0, 0)),
        pl.BlockSpec((1, _D), lambda i: (0, 0)),
        pl.BlockSpec((_D, _D), lambda i: (0, 0)),
        pl.BlockSpec((1, _D), lambda i: (0, 0)),
    ],
    out_specs=[
        pl.BlockSpec((_BLK, _D), lambda i: (i, 0)),
        pl.BlockSpec((1, _D), lambda i: (0, 0)),
        pl.BlockSpec((1, _D), lambda i: (0, 0)),
    ],
    out_shape=[
        jax.ShapeDtypeStruct((_NP, _D), jnp.float32),
        jax.ShapeDtypeStruct((1, _D), jnp.float32),
        jax.ShapeDtypeStruct((1, _D), jnp.float32),
    ],
    scratch_shapes=[
        pltpu.VMEM((1, _D), jnp.float32),
        pltpu.VMEM((1, _D), jnp.float32),
    ],
)


def _norm_body(z2_ref, ssum_ref, ssq_ref, gamma_ref, beta_ref, h_ref):
    m = ssum_ref[...] * (1.0 / _N)
    v = ssq_ref[...] * (1.0 / _N) - m * m
    g = gamma_ref[...] * lax.rsqrt(v + 1e-5)
    h_ref[...] = (z2_ref[...] - m) * g + beta_ref[...]


_norm_call = pl.pallas_call(
    _norm_body,
    grid=(_G,),
    in_specs=[
        pl.BlockSpec((_BLK, _D), lambda i: (i, 0)),
        pl.BlockSpec((1, _D), lambda i: (0, 0)),
        pl.BlockSpec((1, _D), lambda i: (0, 0)),
        pl.BlockSpec((1, _D), lambda i: (0, 0)),
        pl.BlockSpec((1, _D), lambda i: (0, 0)),
    ],
    out_specs=pl.BlockSpec((_BLK, _D), lambda i: (i, 0)),
    out_shape=jax.ShapeDtypeStruct((_NP, _D), jnp.float32),
)


def _pool_body(z2_ref, ssum_ref, ssq_ref, gamma_ref, beta_ref,
               batch_ref, w1_ref, b1_ref, w2_ref, b2_ref,
               hp_ref, lg_ref, hacc, cacc):
    i = pl.program_id(0)

    @pl.when(i == 0)
    def _():
        hacc[...] = jnp.zeros_like(hacc)
        cacc[...] = jnp.zeros_like(cacc)

    m = ssum_ref[...] * (1.0 / _N)
    v = ssq_ref[...] * (1.0 / _N) - m * m
    g = gamma_ref[...] * lax.rsqrt(v + 1e-5)
    h = (z2_ref[...] - m) * g + beta_ref[...]
    bt = batch_ref[0]
    gid = lax.broadcasted_iota(jnp.int32, (_B, _BLK), 0)
    oh = (gid == bt).astype(jnp.float32)
    hacc[...] += jnp.dot(oh, h, preferred_element_type=jnp.float32)
    cacc[...] += jnp.dot(oh, jnp.ones((_BLK, _D), jnp.float32),
                         preferred_element_type=jnp.float32)

    @pl.when(i == pl.num_programs(0) - 1)
    def _():
        hp = hacc[...] / jnp.maximum(cacc[...], 1.0)
        hp_ref[...] = hp
        z = jnp.maximum(
            jnp.dot(hp, w1_ref[...], preferred_element_type=jnp.float32)
            + b1_ref[...], 0.0)
        lg_ref[...] = jnp.dot(z, w2_ref[...],
                              preferred_element_type=jnp.float32) + b2_ref[...]


_pool_call = pl.pallas_call(
    _pool_body,
    grid=(_G,),
    in_specs=[
        pl.BlockSpec((_BLK, _D), lambda i: (i, 0)),
        pl.BlockSpec((1, _D), lambda i: (0, 0)),
        pl.BlockSpec((1, _D), lambda i: (0, 0)),
        pl.BlockSpec((1, _D), lambda i: (0, 0)),
        pl.BlockSpec((1, _D), lambda i: (0, 0)),
        pl.BlockSpec((1, 1, _BLK), lambda i: (i, 0, 0)),
        pl.BlockSpec((_D, _D), lambda i: (0, 0)),
        pl.BlockSpec((1, _D), lambda i: (0, 0)),
        pl.BlockSpec((_D, _D), lambda i: (0, 0)),
        pl.BlockSpec((1, _D), lambda i: (0, 0)),
    ],
    out_specs=[
        pl.BlockSpec((_B, _D), lambda i: (0, 0)),
        pl.BlockSpec((_B, _D), lambda i: (0, 0)),
    ],
    out_shape=[
        jax.ShapeDtypeStruct((_B, _D), jnp.float32),
        jax.ShapeDtypeStruct((_B, _D), jnp.float32),
    ],
    scratch_shapes=[
        pltpu.VMEM((_B, _D), jnp.float32),
        pltpu.VMEM((_B, _D), jnp.float32),
    ],
)


def kernel(x, edge_index, batch, params):
    f32 = jnp.float32
    src = edge_index[0]
    dst = edge_index[1]
    pad_e = _EP - _E
    srcp = jnp.concatenate([src, jnp.zeros((pad_e,), jnp.int32)]).reshape(-1, 128)
    dst_pad = _N + (jnp.arange(pad_e, dtype=jnp.int32) % (_NP - _N))
    dstp = jnp.concatenate([dst, dst_pad]).reshape(-1, 128)
    zeros_slab = jnp.zeros((_RPS, _D), f32)
    batchp = jnp.concatenate(
        [batch, jnp.full((_NP - _N,), _B, jnp.int32)]).reshape(_G, 1, _BLK)

    sc_agg = _make_sc_agg()
    h = jnp.pad(x, ((0, _NP - _N), (0, 0)))
    for li, lp in enumerate(params['layers']):
        aggs = sc_agg(h, srcp, dstp, zeros_slab)
        z2, ssum, ssq = _mlp_call(
            lp['eps'].reshape(1), aggs, h,
            lp['W1'], lp['b1'].reshape(1, _D),
            lp['W2'], lp['b2'].reshape(1, _D))
        if li < 2:
            h = _norm_call(z2, ssum, ssq,
                           lp['gamma'].reshape(1, _D),
                           lp['beta'].reshape(1, _D))

    lp3 = params['layers'][2]
    w2p = jnp.pad(params['lin2_W'], ((0, 0), (0, _D - _C)))
    b2p = jnp.pad(params['lin2_b'], (0, _D - _C)).reshape(1, _D)
    hp, lg = _pool_call(z2, ssum, ssq, lp3['gamma'].reshape(1, _D),
                        lp3['beta'].reshape(1, _D), batchp, params['lin1_W'],
                        params['lin1_b'].reshape(1, _D), w2p, b2p)
    return hp, lg[:, :_C]


# speedup vs baseline: 1.0107x; 1.0107x over previous
import jax
import jax.numpy as jnp
from jax import lax
from jax.experimental import pallas as pl
from jax.experimental.pallas import tpu as pltpu
from jax.experimental.pallas import tpu_sc as plsc

_N = 10000
_NP = 10240
_E = 320000
_D = 128
_B = 64
_C = 10

_NC = 2
_NS = 16
_KC = 80
_KG = 40
_EP = _NC * _NS * _KC * 128
_RPS = _NP // _NS

_BLK = 1024
_G = _NP // _BLK


def _sc_agg_body(h_hbm, src_hbm, dst_hbm, zero_hbm, out_hbm,
                 srcv, dstv, rows, acc, gs0, gs1):
    cid = lax.axis_index("c")
    sid = lax.axis_index("s")
    w = sid * _NC + cid
    gs = (gs0, gs1)

    def wait_gather(b):
        pltpu.make_async_copy(h_hbm.at[pl.ds(0, 128)], rows.at[b], gs[b]).wait()

    pltpu.sync_copy(zero_hbm, acc.at[pl.ds(sid * _RPS, _RPS)])
    plsc.subcore_barrier()

    for g in range(_KC // _KG):
        base = w * _KC + g * _KG
        pltpu.sync_copy(src_hbm.at[pl.ds(base, _KG)], srcv)
        pltpu.sync_copy(dst_hbm.at[pl.ds(base, _KG)], dstv)
        pltpu.async_copy(h_hbm.at[srcv.at[0]], rows.at[0], gs[0])

        def pair(j2, carry):
            for b in range(2):
                jj = 2 * j2 + b

                @pl.when(jj + 1 < _KG)
                def _():
                    pltpu.async_copy(h_hbm.at[srcv.at[jj + 1]],
                                     rows.at[1 - b], gs[1 - b])

                wait_gather(b)
                pltpu.sync_copy(rows.at[b], acc.at[dstv.at[jj]], add=True)
            return carry

        lax.fori_loop(0, _KG // 2, pair, 0)

    plsc.subcore_barrier()
    pltpu.sync_copy(acc.at[pl.ds(sid * _RPS, _RPS)],
                    out_hbm.at[cid, pl.ds(sid * _RPS, _RPS)])


def _make_sc_agg():
    return pl.kernel(
        _sc_agg_body,
        out_type=jax.ShapeDtypeStruct((_NC, _NP, _D), jnp.float32),
        mesh=plsc.VectorSubcoreMesh(core_axis_name="c", subcore_axis_name="s"),
        scratch_types=[
            pltpu.VMEM((_KG, 128), jnp.int32),
            pltpu.VMEM((_KG, 128), jnp.int32),
            pltpu.VMEM((2, 128, _D), jnp.float32),
            pltpu.VMEM_SHARED((_NP, _D), jnp.float32),
            pltpu.SemaphoreType.DMA,
            pltpu.SemaphoreType.DMA,
        ],
    )


def _mlp_body(eps_ref, agg_ref, h_ref, w1_ref, b1_ref, w2_ref, b2_ref,
              z2_ref, ssum_ref, ssq_ref, sacc, qacc):
    i = pl.program_id(0)
    out = agg_ref[0] + agg_ref[1] + (1.0 + eps_ref[0]) * h_ref[...]
    z1 = jnp.maximum(
        jnp.dot(out, w1_ref[...], preferred_element_type=jnp.float32)
        + b1_ref[...], 0.0)
    z2 = jnp.maximum(
        jnp.dot(z1, w2_ref[...], preferred_element_type=jnp.float32)
        + b2_ref[...], 0.0)
    z2_ref[...] = z2

    row = i * _BLK + lax.broadcasted_iota(jnp.int32, (_BLK, 1), 0)
    z2m = jnp.where(row < _N, z2, 0.0)

    @pl.when(i == 0)
    def _():
        sacc[...] = jnp.zeros_like(sacc)
        qacc[...] = jnp.zeros_like(qacc)

    sacc[...] += jnp.sum(z2m, axis=0, keepdims=True)
    qacc[...] += jnp.sum(z2m * z2m, axis=0, keepdims=True)

    @pl.when(i == pl.num_programs(0) - 1)
    def _():
        ssum_ref[...] = sacc[...]
        ssq_ref[...] = qacc[...]


_mlp_call = pl.pallas_call(
    _mlp_body,
    grid=(_G,),
    in_specs=[
        pl.BlockSpec(memory_space=pltpu.SMEM),
        pl.BlockSpec((_NC, _BLK, _D), lambda i: (0, i, 0)),
        pl.BlockSpec((_BLK, _D), lambda i: (i, 0)),
        pl.BlockSpec((_D, _D), lambda i: (0, 0)),
        pl.BlockSpec((1, _D), lambda i: (0, 0)),
        pl.BlockSpec((_D, _D), lambda i: (0, 0)),
        pl.BlockSpec((1, _D), lambda i: (0, 0)),
    ],
    out_specs=[
        pl.BlockSpec((_BLK, _D), lambda i: (i, 0)),
        pl.BlockSpec((1, _D), lambda i: (0, 0)),
        pl.BlockSpec((1, _D), lambda i: (0, 0)),
    ],
    out_shape=[
        jax.ShapeDtypeStruct((_NP, _D), jnp.float32),
        jax.ShapeDtypeStruct((1, _D), jnp.float32),
        jax.ShapeDtypeStruct((1, _D), jnp.float32),
    ],
    scratch_shapes=[
        pltpu.VMEM((1, _D), jnp.float32),
        pltpu.VMEM((1, _D), jnp.float32),
    ],
)


def _norm_body(z2_ref, ssum_ref, ssq_ref, gamma_ref, beta_ref, h_ref):
    m = ssum_ref[...] * (1.0 / _N)
    v = ssq_ref[...] * (1.0 / _N) - m * m
    g = gamma_ref[...] * lax.rsqrt(v + 1e-5)
    h_ref[...] = (z2_ref[...] - m) * g + beta_ref[...]


_norm_call = pl.pallas_call(
    _norm_body,
    grid=(_G,),
    in_specs=[
        pl.BlockSpec((_BLK, _D), lambda i: (i, 0)),
        pl.BlockSpec((1, _D), lambda i: (0, 0)),
        pl.BlockSpec((1, _D), lambda i: (0, 0)),
        pl.BlockSpec((1, _D), lambda i: (0, 0)),
        pl.BlockSpec((1, _D), lambda i: (0, 0)),
    ],
    out_specs=pl.BlockSpec((_BLK, _D), lambda i: (i, 0)),
    out_shape=jax.ShapeDtypeStruct((_NP, _D), jnp.float32),
)


def _pool_body(z2_ref, ssum_ref, ssq_ref, gamma_ref, beta_ref,
               batch_ref, w1_ref, b1_ref, w2_ref, b2_ref,
               hp_ref, lg_ref, hacc, cacc):
    i = pl.program_id(0)

    @pl.when(i == 0)
    def _():
        hacc[...] = jnp.zeros_like(hacc)
        cacc[...] = jnp.zeros_like(cacc)

    m = ssum_ref[...] * (1.0 / _N)
    v = ssq_ref[...] * (1.0 / _N) - m * m
    g = gamma_ref[...] * lax.rsqrt(v + 1e-5)
    h = (z2_ref[...] - m) * g + beta_ref[...]
    bt = batch_ref[0]
    gid = lax.broadcasted_iota(jnp.int32, (_B, _BLK), 0)
    oh = (gid == bt).astype(jnp.float32)
    hacc[...] += jnp.dot(oh, h, preferred_element_type=jnp.float32)
    cacc[...] += jnp.dot(oh, jnp.ones((_BLK, _D), jnp.float32),
                         preferred_element_type=jnp.float32)

    @pl.when(i == pl.num_programs(0) - 1)
    def _():
        hp = hacc[...] / jnp.maximum(cacc[...], 1.0)
        hp_ref[...] = hp
        z = jnp.maximum(
            jnp.dot(hp, w1_ref[...], preferred_element_type=jnp.float32)
            + b1_ref[...], 0.0)
        lg_ref[...] = jnp.dot(z, w2_ref[...],
                              preferred_element_type=jnp.float32) + b2_ref[...]


_pool_call = pl.pallas_call(
    _pool_body,
    grid=(_G,),
    in_specs=[
        pl.BlockSpec((_BLK, _D), lambda i: (i, 0)),
        pl.BlockSpec((1, _D), lambda i: (0, 0)),
        pl.BlockSpec((1, _D), lambda i: (0, 0)),
        pl.BlockSpec((1, _D), lambda i: (0, 0)),
        pl.BlockSpec((1, _D), lambda i: (0, 0)),
        pl.BlockSpec((1, 1, _BLK), lambda i: (i, 0, 0)),
        pl.BlockSpec((_D, _D), lambda i: (0, 0)),
        pl.BlockSpec((1, _D), lambda i: (0, 0)),
        pl.BlockSpec((_D, _D), lambda i: (0, 0)),
        pl.BlockSpec((1, _D), lambda i: (0, 0)),
    ],
    out_specs=[
        pl.BlockSpec((_B, _D), lambda i: (0, 0)),
        pl.BlockSpec((_B, _D), lambda i: (0, 0)),
    ],
    out_shape=[
        jax.ShapeDtypeStruct((_B, _D), jnp.float32),
        jax.ShapeDtypeStruct((_B, _D), jnp.float32),
    ],
    scratch_shapes=[
        pltpu.VMEM((_B, _D), jnp.float32),
        pltpu.VMEM((_B, _D), jnp.float32),
    ],
)


def _mlpn_body(eps_ref, agg_ref, h_ref, w1_ref, b1_ref, w2_ref, b2_ref,
               gamma_ref, beta_ref, h_out_ref, z2buf, sacc, qacc):
    i = pl.program_id(0)
    blk = lax.rem(i, _G)

    @pl.when(i == 0)
    def _():
        sacc[...] = jnp.zeros_like(sacc)
        qacc[...] = jnp.zeros_like(qacc)

    @pl.when(i < _G)
    def _():
        out = agg_ref[0] + agg_ref[1] + (1.0 + eps_ref[0]) * h_ref[...]
        z1 = jnp.maximum(
            jnp.dot(out, w1_ref[...], preferred_element_type=jnp.float32)
            + b1_ref[...], 0.0)
        z2 = jnp.maximum(
            jnp.dot(z1, w2_ref[...], preferred_element_type=jnp.float32)
            + b2_ref[...], 0.0)
        z2buf[pl.ds(blk * _BLK, _BLK), :] = z2
        row = blk * _BLK + lax.broadcasted_iota(jnp.int32, (_BLK, 1), 0)
        z2m = jnp.where(row < _N, z2, 0.0)
        sacc[...] += jnp.sum(z2m, axis=0, keepdims=True)
        qacc[...] += jnp.sum(z2m * z2m, axis=0, keepdims=True)

    @pl.when(i >= _G)
    def _():
        m = sacc[...] * (1.0 / _N)
        v = qacc[...] * (1.0 / _N) - m * m
        g = gamma_ref[...] * lax.rsqrt(v + 1e-5)
        z2 = z2buf[pl.ds(blk * _BLK, _BLK), :]
        h_out_ref[...] = (z2 - m) * g + beta_ref[...]


_mlpn_call = pl.pallas_call(
    _mlpn_body,
    grid=(2 * _G,),
    in_specs=[
        pl.BlockSpec(memory_space=pltpu.SMEM),
        pl.BlockSpec((_NC, _BLK, _D), lambda i: (0, i % _G, 0)),
        pl.BlockSpec((_BLK, _D), lambda i: (i % _G, 0)),
        pl.BlockSpec((_D, _D), lambda i: (0, 0)),
        pl.BlockSpec((1, _D), lambda i: (0, 0)),
        pl.BlockSpec((_D, _D), lambda i: (0, 0)),
        pl.BlockSpec((1, _D), lambda i: (0, 0)),
        pl.BlockSpec((1, _D), lambda i: (0, 0)),
        pl.BlockSpec((1, _D), lambda i: (0, 0)),
    ],
    out_specs=pl.BlockSpec((_BLK, _D), lambda i: (i % _G, 0)),
    out_shape=jax.ShapeDtypeStruct((_NP, _D), jnp.float32),
    scratch_shapes=[
        pltpu.VMEM((_NP, _D), jnp.float32),
        pltpu.VMEM((1, _D), jnp.float32),
        pltpu.VMEM((1, _D), jnp.float32),
    ],
)


def kernel(x, edge_index, batch, params):
    f32 = jnp.float32
    src = edge_index[0]
    dst = edge_index[1]
    pad_e = _EP - _E
    srcp = jnp.concatenate([src, jnp.zeros((pad_e,), jnp.int32)]).reshape(-1, 128)
    dst_pad = _N + (jnp.arange(pad_e, dtype=jnp.int32) % (_NP - _N))
    dstp = jnp.concatenate([dst, dst_pad]).reshape(-1, 128)
    zeros_slab = jnp.zeros((_RPS, _D), f32)
    batchp = jnp.concatenate(
        [batch, jnp.full((_NP - _N,), _B, jnp.int32)]).reshape(_G, 1, _BLK)

    sc_agg = _make_sc_agg()
    h = jnp.pad(x, ((0, _NP - _N), (0, 0)))
    for li, lp in enumerate(params['layers']):
        aggs = sc_agg(h, srcp, dstp, zeros_slab)
        if li < 2:
            h = _mlpn_call(
                lp['eps'].reshape(1), aggs, h,
                lp['W1'], lp['b1'].reshape(1, _D),
                lp['W2'], lp['b2'].reshape(1, _D),
                lp['gamma'].reshape(1, _D), lp['beta'].reshape(1, _D))
        else:
            z2, ssum, ssq = _mlp_call(
                lp['eps'].reshape(1), aggs, h,
                lp['W1'], lp['b1'].reshape(1, _D),
                lp['W2'], lp['b2'].reshape(1, _D))

    lp3 = params['layers'][2]
    w2p = jnp.pad(params['lin2_W'], ((0, 0), (0, _D - _C)))
    b2p = jnp.pad(params['lin2_b'], (0, _D - _C)).reshape(1, _D)
    hp, lg = _pool_call(z2, ssum, ssq, lp3['gamma'].reshape(1, _D),
                        lp3['beta'].reshape(1, _D), batchp, params['lin1_W'],
                        params['lin1_b'].reshape(1, _D), w2p, b2p)
    return hp, lg[:, :_C]
